# baseline (speedup 1.0000x reference)
import functools

import jax
import jax.numpy as jnp
from jax import lax
from jax.experimental import pallas as pl
from jax.experimental.pallas import tpu as pltpu
from jax.experimental.pallas import tpu_sc as plsc

NTOK = 16384
D = 4096
SQRT_K = 8
NE = SQRT_K * SQRT_K
TOP_K = 8
BLK = 1024

NC = 2
NS = 16
NW = NC * NS
TPW = NTOK // (NW * 4)
CH = 16
G = 2


def _tc_scores_body(x_ref, wct_ref, scores_ref):
    s = jnp.dot(
        x_ref[...].astype(jnp.bfloat16),
        wct_ref[...].astype(jnp.bfloat16),
        preferred_element_type=jnp.float32,
    )
    row = lax.broadcasted_iota(jnp.int32, (2 * SQRT_K, NE), 0)
    col = lax.broadcasted_iota(jnp.int32, (2 * SQRT_K, NE), 1)
    exp1 = ((row < SQRT_K) & ((col // SQRT_K) == row)).astype(jnp.float32)
    exp2 = ((row >= SQRT_K) & ((col % SQRT_K) == (row - SQRT_K))).astype(
        jnp.float32
    )
    rep1 = jnp.dot(s, exp1, preferred_element_type=jnp.float32,
                   precision=lax.Precision.HIGHEST)
    tile2 = jnp.dot(s, exp2, preferred_element_type=jnp.float32,
                    precision=lax.Precision.HIGHEST)
    scores_ref[...] = rep1 + tile2


NCHUNK = 4
CT = NTOK // NCHUNK


def _tc_scores(x, wct, ci):
    return pl.pallas_call(
        _tc_scores_body,
        grid=(CT // BLK,),
        in_specs=[
            pl.BlockSpec((BLK, D), lambda i: (i + ci * (CT // BLK), 0)),
            pl.BlockSpec((D, 2 * SQRT_K), lambda i: (0, 0)),
        ],
        out_specs=pl.BlockSpec((BLK, NE), lambda i: (i, 0)),
        out_shape=jax.ShapeDtypeStruct((CT, NE), jnp.float32),
    )(x, wct)


def _sc_route_body(scores_hbm, ltau_hbm, idx_hbm, gates_hbm, sbuf, ibuf, gbuf,
                   ltv):
    wid = lax.axis_index("s") * NC + lax.axis_index("c")
    pltpu.sync_copy(scores_hbm.at[pl.ds(wid * (TPW * NE), TPW * NE)], sbuf)
    pltpu.sync_copy(ltau_hbm, ltv)
    tau = jnp.exp(ltv[...])
    lanes = lax.iota(jnp.int32, 16)

    @plsc.parallel_loop(0, TPW // (CH * G), 1)
    def chunk(c):
        saddr = [((c * G + g) * CH + lanes) * NE for g in range(G)]
        oaddr = [((c * G + g) * CH + lanes) * TOP_K for g in range(G)]
        neg = jnp.full((16,), -jnp.inf, jnp.float32)
        vals = [[neg] * TOP_K for _ in range(G)]
        idxs = [[neg] * TOP_K for _ in range(G)]
        for e in range(NE):
            ern = (e + lanes) & (NE - 1)
            ef = ern.astype(jnp.float32)
            vs = [plsc.load_gather(sbuf, [saddr[g] + ern]) for g in range(G)]
            for g in range(G):
                v = vs[g]
                cmps = [v > vals[g][r] for r in range(TOP_K)]
                nv = [jnp.where(cmps[0], v, vals[g][0])]
                ni = [jnp.where(cmps[0], ef, idxs[g][0])]
                for r in range(1, TOP_K):
                    nv.append(jnp.where(
                        cmps[r], jnp.where(cmps[r - 1], vals[g][r - 1], v),
                        vals[g][r]))
                    ni.append(jnp.where(
                        cmps[r], jnp.where(cmps[r - 1], idxs[g][r - 1], ef),
                        idxs[g][r]))
                vals[g] = nv
                idxs[g] = ni
        for g in range(G):
            m = vals[g][0]
            ex = [jnp.exp((vals[g][r] - m) / tau) for r in range(TOP_K)]
            tot = ex[0]
            for r in range(1, TOP_K):
                tot = tot + ex[r]
            for r in range(TOP_K):
                plsc.store_scatter(ibuf, [oaddr[g] + r],
                                   idxs[g][r].astype(jnp.int32))
                plsc.store_scatter(gbuf, [oaddr[g] + r], ex[r] / tot)
    pltpu.sync_copy(ibuf, idx_hbm.at[pl.ds(wid * (TPW * TOP_K), TPW * TOP_K)])
    pltpu.sync_copy(gbuf,
                    gates_hbm.at[pl.ds(wid * (TPW * TOP_K), TPW * TOP_K)])


@functools.partial(
    pl.kernel,
    out_type=[
        jax.ShapeDtypeStruct((CT * TOP_K,), jnp.int32),
        jax.ShapeDtypeStruct((CT * TOP_K,), jnp.float32),
    ],
    mesh=plsc.VectorSubcoreMesh(core_axis_name="c", subcore_axis_name="s"),
    compiler_params=pltpu.CompilerParams(needs_layout_passes=False),
    scratch_types=[
        pltpu.VMEM((TPW * NE,), jnp.float32),
        pltpu.VMEM((TPW * TOP_K,), jnp.int32),
        pltpu.VMEM((TPW * TOP_K,), jnp.float32),
        pltpu.VMEM((16,), jnp.float32),
    ],
)
def _sc_route(scores_hbm, ltau_hbm, idx_hbm, gates_hbm, sbuf, ibuf, gbuf, ltv):
    _sc_route_body(scores_hbm, ltau_hbm, idx_hbm, gates_hbm, sbuf, ibuf, gbuf,
                   ltv)


@jax.jit
def kernel(x, W1, W2, log_tau):
    wct = jnp.concatenate([W1, W2], axis=0).T
    ltau16 = jnp.full((16,), log_tau, jnp.float32)
    sc_list, idx_list, gate_list = [], [], []
    for ci in range(NCHUNK):
        sc_i = _tc_scores(x, wct, ci)
        idx_i, gates_i = _sc_route(sc_i.reshape(CT * NE), ltau16)
        sc_list.append(sc_i)
        idx_list.append(idx_i.reshape(CT, TOP_K))
        gate_list.append(gates_i.reshape(CT, TOP_K))
    return (jnp.concatenate(idx_list), jnp.concatenate(gate_list),
            jnp.concatenate(sc_list))


# speedup vs baseline: 1.0062x; 1.0062x over previous
import functools

import jax
import jax.numpy as jnp
from jax import lax
from jax.experimental import pallas as pl
from jax.experimental.pallas import tpu as pltpu
from jax.experimental.pallas import tpu_sc as plsc

NTOK = 16384
D = 4096
SQRT_K = 8
NE = SQRT_K * SQRT_K
TOP_K = 8
BLK = 1024

NC = 2
NS = 16
NW = NC * NS
TPW = NTOK // (NW * 4)
CH = 16
G = 2


def _tc_scores_body(x_ref, wct_ref, scores_ref):
    s = jnp.dot(
        x_ref[...].astype(jnp.bfloat16),
        wct_ref[...].astype(jnp.bfloat16),
        preferred_element_type=jnp.float32,
    )
    row = lax.broadcasted_iota(jnp.int32, (2 * SQRT_K, NE), 0)
    col = lax.broadcasted_iota(jnp.int32, (2 * SQRT_K, NE), 1)
    exp1 = ((row < SQRT_K) & ((col // SQRT_K) == row)).astype(jnp.float32)
    exp2 = ((row >= SQRT_K) & ((col % SQRT_K) == (row - SQRT_K))).astype(
        jnp.float32
    )
    rep1 = jnp.dot(s, exp1, preferred_element_type=jnp.float32,
                   precision=lax.Precision.HIGHEST)
    tile2 = jnp.dot(s, exp2, preferred_element_type=jnp.float32,
                    precision=lax.Precision.HIGHEST)
    scores_ref[...] = rep1 + tile2


NCHUNK = 4
CT = NTOK // NCHUNK


def _tc_scores(x, wct, ci):
    return pl.pallas_call(
        _tc_scores_body,
        grid=(CT // BLK,),
        in_specs=[
            pl.BlockSpec((BLK, D), lambda i: (i + ci * (CT // BLK), 0)),
            pl.BlockSpec((D, 2 * SQRT_K), lambda i: (0, 0)),
        ],
        out_specs=pl.BlockSpec((BLK, NE), lambda i: (i, 0)),
        out_shape=jax.ShapeDtypeStruct((CT, NE), jnp.float32),
    )(x, wct)


def _sc_route_body(scores_hbm, ltau_hbm, idx_hbm, gates_hbm, sbuf, ibuf, gbuf,
                   ltv):
    wid = lax.axis_index("s") * NC + lax.axis_index("c")
    pltpu.sync_copy(scores_hbm.at[pl.ds(wid * (TPW * NE), TPW * NE)], sbuf)
    pltpu.sync_copy(ltau_hbm, ltv)
    tau = jnp.exp(ltv[...])
    lanes = lax.iota(jnp.int32, 16)

    @plsc.parallel_loop(0, TPW // (CH * G), 1)
    def chunk(c):
        saddr = [((c * G + g) * CH + lanes) * NE for g in range(G)]
        oaddr = [((c * G + g) * CH + lanes) * TOP_K for g in range(G)]
        neg = jnp.full((16,), -jnp.inf, jnp.float32)
        vals = [[neg] * TOP_K for _ in range(G)]
        idxs = [[neg] * TOP_K for _ in range(G)]
        for e in range(NE):
            ern = (e + lanes) & (NE - 1)
            ef = ern.astype(jnp.float32)
            vs = [plsc.load_gather(sbuf, [saddr[g] + ern]) for g in range(G)]
            for g in range(G):
                v = vs[g]
                cmps = [v > vals[g][r] for r in range(TOP_K)]
                nv = [jnp.where(cmps[0], v, vals[g][0])]
                ni = [jnp.where(cmps[0], ef, idxs[g][0])]
                for r in range(1, TOP_K):
                    nv.append(jnp.where(
                        cmps[r], jnp.where(cmps[r - 1], vals[g][r - 1], v),
                        vals[g][r]))
                    ni.append(jnp.where(
                        cmps[r], jnp.where(cmps[r - 1], idxs[g][r - 1], ef),
                        idxs[g][r]))
                vals[g] = nv
                idxs[g] = ni
        for g in range(G):
            m = vals[g][0]
            ex = [jnp.exp((vals[g][r] - m) / tau) for r in range(TOP_K)]
            tot = ex[0]
            for r in range(1, TOP_K):
                tot = tot + ex[r]
            for r in range(TOP_K):
                plsc.store_scatter(ibuf, [oaddr[g] + r],
                                   idxs[g][r].astype(jnp.int32))
                plsc.store_scatter(gbuf, [oaddr[g] + r], ex[r] / tot)
    pltpu.sync_copy(ibuf, idx_hbm.at[pl.ds(wid * (TPW * TOP_K), TPW * TOP_K)])
    pltpu.sync_copy(gbuf,
                    gates_hbm.at[pl.ds(wid * (TPW * TOP_K), TPW * TOP_K)])


@functools.partial(
    pl.kernel,
    out_type=[
        jax.ShapeDtypeStruct((CT * TOP_K,), jnp.int32),
        jax.ShapeDtypeStruct((CT * TOP_K,), jnp.float32),
    ],
    mesh=plsc.VectorSubcoreMesh(core_axis_name="c", subcore_axis_name="s"),
    compiler_params=pltpu.CompilerParams(needs_layout_passes=False),
    scratch_types=[
        pltpu.VMEM((TPW * NE,), jnp.float32),
        pltpu.VMEM((TPW * TOP_K,), jnp.int32),
        pltpu.VMEM((TPW * TOP_K,), jnp.float32),
        pltpu.VMEM((16,), jnp.float32),
    ],
)
def _sc_route(scores_hbm, ltau_hbm, idx_hbm, gates_hbm, sbuf, ibuf, gbuf, ltv):
    _sc_route_body(scores_hbm, ltau_hbm, idx_hbm, gates_hbm, sbuf, ibuf, gbuf,
                   ltv)


@jax.jit
def kernel(x, W1, W2, log_tau):
    wct = jnp.concatenate([W1, W2], axis=0).T
    ltau16 = jnp.full((16,), log_tau, jnp.float32)
    sc_list, idx_list, gate_list = [], [], []
    prev = None
    for ci in range(NCHUNK):
        sc_i = _tc_scores(x, wct, ci)
        sc_list.append(sc_i)
        if prev is not None:
            idx_p, gates_p = _sc_route(prev.reshape(CT * NE), ltau16)
            idx_list.append(idx_p.reshape(CT, TOP_K))
            gate_list.append(gates_p.reshape(CT, TOP_K))
        prev = sc_i
    idx_p, gates_p = _sc_route(prev.reshape(CT * NE), ltau16)
    idx_list.append(idx_p.reshape(CT, TOP_K))
    gate_list.append(gates_p.reshape(CT, TOP_K))
    return (jnp.concatenate(idx_list), jnp.concatenate(gate_list),
            jnp.concatenate(sc_list))


# speedup vs baseline: 1.3237x; 1.3156x over previous
import jax
import jax.numpy as jnp
from jax import lax
from jax.experimental import pallas as pl
from jax.experimental.pallas import tpu as pltpu

NTOK = 16384
D = 4096
SQRT_K = 8
NE = SQRT_K * SQRT_K
TOP_K = 8
BLK = 1024


def _router_body(log_tau_ref, x_ref, wct_ref, idxt_ref, gatest_ref,
                 scores_ref):
    s = jnp.dot(
        x_ref[...].astype(jnp.bfloat16),
        wct_ref[...].astype(jnp.bfloat16),
        preferred_element_type=jnp.float32,
    )
    row = lax.broadcasted_iota(jnp.int32, (2 * SQRT_K, NE), 0)
    col = lax.broadcasted_iota(jnp.int32, (2 * SQRT_K, NE), 1)
    exp1 = ((row < SQRT_K) & ((col // SQRT_K) == row)).astype(jnp.float32)
    exp2 = ((row >= SQRT_K) & ((col % SQRT_K) == (row - SQRT_K))).astype(
        jnp.float32
    )
    rep1 = jnp.dot(s, exp1, preferred_element_type=jnp.float32,
                   precision=lax.Precision.HIGHEST)
    tile2 = jnp.dot(s, exp2, preferred_element_type=jnp.float32,
                    precision=lax.Precision.HIGHEST)
    scores_ref[...] = rep1 + tile2

    st = jnp.transpose(s)
    scorest = (
        jnp.dot(exp1.T, st, preferred_element_type=jnp.float32,
                precision=lax.Precision.HIGHEST)
        + jnp.dot(exp2.T, st, preferred_element_type=jnp.float32,
                  precision=lax.Precision.HIGHEST)
    )

    tau = jnp.exp(log_tau_ref[0, 0])
    rowf = lax.broadcasted_iota(jnp.int32, (NE, BLK), 0).astype(jnp.float32)
    row8 = lax.broadcasted_iota(jnp.int32, (TOP_K, BLK), 0)
    work = scorest
    vals8 = jnp.zeros((TOP_K, BLK), jnp.float32)
    idx8 = jnp.zeros((TOP_K, BLK), jnp.float32)
    for k in range(TOP_K):
        m = jnp.max(work, axis=0, keepdims=True)
        pick = jnp.min(jnp.where(work == m, rowf, jnp.float32(NE)), axis=0,
                       keepdims=True)
        vals8 = jnp.where(row8 == k, m, vals8)
        idx8 = jnp.where(row8 == k, pick, idx8)
        work = jnp.where(rowf == pick, -jnp.inf, work)

    mx = jnp.max(vals8, axis=0, keepdims=True)
    ex = jnp.exp((vals8 - mx) / tau)
    gatest_ref[...] = ex / jnp.sum(ex, axis=0, keepdims=True)
    idxt_ref[...] = idx8.astype(jnp.int32)


@jax.jit
def kernel(x, W1, W2, log_tau):
    wct = jnp.concatenate([W1, W2], axis=0).T
    lt = log_tau.reshape(1, 1)
    grid = NTOK // BLK
    idxt, gatest, scores = pl.pallas_call(
        _router_body,
        grid=(grid,),
        in_specs=[
            pl.BlockSpec(memory_space=pltpu.SMEM),
            pl.BlockSpec((BLK, D), lambda i: (i, 0)),
            pl.BlockSpec((D, 2 * SQRT_K), lambda i: (0, 0)),
        ],
        out_specs=[
            pl.BlockSpec((TOP_K, BLK), lambda i: (0, i)),
            pl.BlockSpec((TOP_K, BLK), lambda i: (0, i)),
            pl.BlockSpec((BLK, NE), lambda i: (i, 0)),
        ],
        out_shape=[
            jax.ShapeDtypeStruct((TOP_K, NTOK), jnp.int32),
            jax.ShapeDtypeStruct((TOP_K, NTOK), jnp.float32),
            jax.ShapeDtypeStruct((NTOK, NE), jnp.float32),
        ],
    )(lt, x, wct)
    return idxt.T, gatest.T, scores
